# Initial kernel scaffold; baseline (speedup 1.0000x reference)
import jax
import jax.numpy as jnp
from jax.experimental import pallas as pl


def kernel(x_entity, x_snapshot, ee_src, ee_dst, es_src, es_dst, ss_src, ss_dst, W_ee, b_ee, W_es, b_es, W_ss, b_ss, W_skip_ent, b_skip_ent, W_skip_snap, b_skip_snap, W_trans_ent, b_trans_ent, W_trans_snap, b_trans_snap):
    raise NotImplementedError("write your pallas kernel here")


# speedup vs baseline: 4.4221x; 4.4221x over previous
import jax
import jax.numpy as jnp
from jax import lax
from jax.experimental import pallas as pl
from jax.experimental.pallas import tpu as pltpu
from jax.experimental.pallas import tpu_sc as plsc

N_ENT = 10000
N_SNAP = 10
D = 128

NC = 2
NS = 16
NW = NC * NS
LANES = 16
CHUNK = 128

HALF = 5120
R_HALF = 6144
R_ACC = 2 * HALF
DUMMY = N_ENT + N_SNAP
ZPT = R_HALF // NS
OPT = HALF // NS
ENT_BLK = 400


def _sc_body(x_hbm, src_hbm, dst_hbm,
             acc_out, deg_out,
             src_v, dst_v, rows_a, rows_b, deg_v,
             acc_sh, sem_a, sem_b):
    c = lax.axis_index("c")
    s = lax.axis_index("s")
    cpt = src_v.shape[0]

    zeros16 = jnp.zeros((LANES,), jnp.float32)
    ones16 = jnp.ones((LANES,), jnp.float32)

    def _zrow(i, _):
        for k in range(D // LANES):
            rows_a[i, pl.ds(k * LANES, LANES)] = zeros16
        return 0
    lax.fori_loop(0, CHUNK, _zrow, 0)

    def _zdeg(i, _):
        deg_v[pl.ds(i * LANES, LANES)] = zeros16
        return 0
    lax.fori_loop(0, R_HALF // LANES, _zdeg, 0)

    for i in range(ZPT // CHUNK):
        pltpu.sync_copy(rows_a, acc_sh.at[pl.ds(s * ZPT + i * CHUNK, CHUNK)])

    pltpu.sync_copy(src_hbm.at[s], src_v)
    pltpu.sync_copy(dst_hbm.at[c, s], dst_v)
    plsc.subcore_barrier()

    def _deg_update(j):
        for k in range(CHUNK // LANES):
            idx = dst_v[j, pl.ds(k * LANES, LANES)]
            plsc.addupdate_scatter(deg_v, [idx], ones16)

    pltpu.async_copy(x_hbm.at[src_v.at[0]], rows_a, sem_a)

    def _pair(j0, issue_next):
        j1 = j0 + 1
        pltpu.make_async_copy(x_hbm.at[src_v.at[j0]], rows_a, sem_a).wait()
        pltpu.async_copy(x_hbm.at[src_v.at[j1]], rows_b, sem_b)
        _deg_update(j0)
        pltpu.sync_copy(rows_a, acc_sh.at[dst_v.at[j0]], add=True)
        pltpu.make_async_copy(x_hbm.at[src_v.at[j1]], rows_b, sem_b).wait()
        if issue_next:
            pltpu.async_copy(x_hbm.at[src_v.at[j1 + 1]], rows_a, sem_a)
        _deg_update(j1)
        pltpu.sync_copy(rows_b, acc_sh.at[dst_v.at[j1]], add=True)

    def _step(t, _):
        _pair(2 * t, True)
        return 0

    lax.fori_loop(0, cpt // 2 - 1, _step, 0)
    _pair(cpt - 2, False)

    pltpu.sync_copy(deg_v, deg_out.at[c * NS + s])
    plsc.subcore_barrier()

    off = 0
    while off < OPT:
        n = min(CHUNK, OPT - off)
        pltpu.sync_copy(acc_sh.at[pl.ds(s * OPT + off, n)],
                        rows_a.at[pl.ds(0, n)])
        pltpu.sync_copy(rows_a.at[pl.ds(0, n)],
                        acc_out.at[pl.ds(c * HALF + s * OPT + off, n)])
        off += n


def _sc_aggregate(x_entity, src3, dst4):
    cpt = src3.shape[1]
    mesh = plsc.VectorSubcoreMesh(core_axis_name="c", subcore_axis_name="s")
    return pl.kernel(
        _sc_body,
        out_type=(
            jax.ShapeDtypeStruct((R_ACC, D), jnp.float32),
            jax.ShapeDtypeStruct((NW, R_HALF), jnp.float32),
        ),
        mesh=mesh,
        compiler_params=pltpu.CompilerParams(needs_layout_passes=False),
        scratch_types=[
            pltpu.VMEM((cpt, CHUNK), jnp.int32),
            pltpu.VMEM((cpt, CHUNK), jnp.int32),
            pltpu.VMEM((CHUNK, D), jnp.float32),
            pltpu.VMEM((CHUNK, D), jnp.float32),
            pltpu.VMEM((R_HALF,), jnp.float32),
            pltpu.VMEM_SHARED((R_HALF, D), jnp.float32),
            pltpu.SemaphoreType.DMA,
            pltpu.SemaphoreType.DMA,
        ],
    )(x_entity, src3, dst4)


def _ent_body(x_ref, acc_ref, deg_ref, wskip_ref, wee_ref, wtrans_ref,
              bskip_ref, bee_ref, btrans_ref, out_ref):
    agg = acc_ref[...]
    d = jnp.sum(deg_ref[0], axis=0)
    d = jnp.maximum(d, 1.0)
    x = x_ref[...]
    h = jnp.dot(x, wskip_ref[...], preferred_element_type=jnp.float32)
    h = h + jnp.dot(agg / d[:, None], wee_ref[...],
                    preferred_element_type=jnp.float32)
    h = h + bskip_ref[...] + bee_ref[...]
    h = jnp.where(h >= 0, h, 0.01 * h)
    out_ref[...] = jnp.dot(h, wtrans_ref[...],
                           preferred_element_type=jnp.float32) + btrans_ref[...]


def _ent_path(x_entity, acc, deg4, w_skip, w_ee, w_trans, b_skip, b_ee, b_trans):
    blk = ENT_BLK
    grid = N_ENT // blk
    wspec = pl.BlockSpec((D, D), lambda i: (0, 0))
    bspec = pl.BlockSpec((1, D), lambda i: (0, 0))
    return pl.pallas_call(
        _ent_body,
        grid=(grid,),
        in_specs=[
            pl.BlockSpec((blk, D), lambda i: (i, 0)),
            pl.BlockSpec((blk, D), lambda i: (i, 0)),
            pl.BlockSpec((1, NS, blk), lambda i: (i, 0, 0)),
            wspec, wspec, wspec, bspec, bspec, bspec,
        ],
        out_specs=pl.BlockSpec((blk, D), lambda i: (i, 0)),
        out_shape=jax.ShapeDtypeStruct((N_ENT, D), jnp.float32),
    )(x_entity, acc, deg4, w_skip, w_ee, w_trans,
      b_skip.reshape(1, D), b_ee.reshape(1, D), b_trans.reshape(1, D))


def _snap_body(xs_ref, acc_ref, deg_ref, ss_src_ref, ss_dst_ref,
               wskip_ref, wes_ref, wss_ref, wtrans_ref,
               bskip_ref, bes_ref, bss_ref, btrans_ref, out_ref):
    m = 16
    aggs = acc_ref[...]
    rowid = lax.broadcasted_iota(jnp.int32, (m, D), 0)
    aggs = jnp.where(rowid < N_SNAP, aggs, 0.0)
    ds_ = jnp.sum(deg_ref[...], axis=0)
    ds_ = jnp.maximum(ds_, 1.0)
    conv_es = jnp.dot(aggs / ds_[:, None], wes_ref[...],
                      preferred_element_type=jnp.float32) + bes_ref[...]
    xs = xs_ref[...]
    h0 = jnp.dot(xs, wskip_ref[...],
                 preferred_element_type=jnp.float32) + bskip_ref[...]
    h0 = h0 + conv_es[:N_SNAP]

    colid = lax.broadcasted_iota(jnp.int32, (m, D), 1)
    a = jnp.zeros((m, D), jnp.float32)
    for e in range(ss_src_ref.shape[0]):
        se = ss_src_ref[e]
        de = ss_dst_ref[e]
        a = a + jnp.where((rowid == de) & (colid == se), 1.0, 0.0)
    h0p = jnp.concatenate([h0, jnp.zeros((D - N_SNAP, D), jnp.float32)], axis=0)
    aggss = jnp.dot(a, h0p, preferred_element_type=jnp.float32)
    degss = jnp.maximum(jnp.sum(a, axis=1), 1.0)
    hs = jnp.dot(aggss / degss[:, None], wss_ref[...],
                 preferred_element_type=jnp.float32) + bss_ref[...]
    hs = jnp.where(hs >= 0, hs, 0.01 * hs)
    res = jnp.dot(hs, wtrans_ref[...],
                  preferred_element_type=jnp.float32) + btrans_ref[...]
    out_ref[...] = res[:N_SNAP]


def _snap_path(x_snapshot, acc_es, deg_es, ss_src, ss_dst,
               w_skip, w_es, w_ss, w_trans, b_skip, b_es, b_ss, b_trans):
    wspec = pl.BlockSpec((D, D), lambda: (0, 0))
    bspec = pl.BlockSpec((1, D), lambda: (0, 0))
    sspec = pl.BlockSpec(memory_space=pltpu.SMEM)
    return pl.pallas_call(
        _snap_body,
        in_specs=[
            pl.BlockSpec((N_SNAP, D), lambda: (0, 0)),
            pl.BlockSpec((16, D), lambda: (0, 0)),
            pl.BlockSpec((NS, 16), lambda: (0, 0)),
            sspec, sspec,
            wspec, wspec, wspec, wspec, bspec, bspec, bspec, bspec,
        ],
        out_specs=pl.BlockSpec((N_SNAP, D), lambda: (0, 0)),
        out_shape=jax.ShapeDtypeStruct((N_SNAP, D), jnp.float32),
    )(x_snapshot, acc_es, deg_es, ss_src, ss_dst,
      w_skip, w_es, w_ss, w_trans,
      b_skip.reshape(1, D), b_es.reshape(1, D), b_ss.reshape(1, D),
      b_trans.reshape(1, D))


def kernel(x_entity, x_snapshot, ee_src, ee_dst, es_src, es_dst, ss_src, ss_dst,
           W_ee, b_ee, W_es, b_es, W_ss, b_ss,
           W_skip_ent, b_skip_ent, W_skip_snap, b_skip_snap,
           W_trans_ent, b_trans_ent, W_trans_snap, b_trans_snap):
    e_total = ee_src.shape[0] + es_src.shape[0]
    e_pad = -(-e_total // (NS * CHUNK * 2)) * (NS * CHUNK * 2)
    npad = e_pad - e_total
    src3 = jnp.concatenate(
        [ee_src, es_src, jnp.zeros((npad,), jnp.int32)]).reshape(NS, -1, CHUNK)
    dstg = jnp.concatenate(
        [ee_dst, es_dst + N_ENT, jnp.full((npad,), DUMMY, jnp.int32)])
    d0 = jnp.where(dstg < HALF, dstg, HALF)
    d1t = dstg - HALF
    d1 = jnp.where((d1t >= 0) & (d1t < HALF), d1t, HALF)
    dst4 = jnp.stack([d0, d1]).reshape(NC, NS, -1, CHUNK)

    acc, deg = _sc_aggregate(x_entity, src3, dst4)
    dglob = (deg.reshape(NC, NS, R_HALF)[:, :, :HALF]
             .transpose(1, 0, 2).reshape(NS, R_ACC))

    deg4 = (dglob[:, :N_ENT].reshape(NS, N_ENT // ENT_BLK, ENT_BLK)
            .transpose(1, 0, 2))
    h_ent = _ent_path(x_entity, acc, deg4, W_skip_ent, W_ee, W_trans_ent,
                      b_skip_ent, b_ee, b_trans_ent)

    acc_es = acc[N_ENT:N_ENT + 16]
    deg_es = dglob[:, N_ENT:N_ENT + 16]
    h_snap = _snap_path(x_snapshot, acc_es, deg_es, ss_src, ss_dst,
                        W_skip_snap, W_es, W_ss, W_trans_snap,
                        b_skip_snap, b_es, b_ss, b_trans_snap)
    return (h_ent, h_snap)


# speedup vs baseline: 4.6239x; 1.0456x over previous
import jax
import jax.numpy as jnp
from jax import lax
from jax.experimental import pallas as pl
from jax.experimental.pallas import tpu as pltpu
from jax.experimental.pallas import tpu_sc as plsc

N_ENT = 10000
N_SNAP = 10
D = 128

NC = 2
NS = 16
NW = NC * NS
LANES = 16
CHUNK = 128

HALF = 5120
R_HALF = 6144
R_ACC = 2 * HALF
DUMMY = N_ENT + N_SNAP
ZPT = R_HALF // NS
OPT = HALF // NS
ENT_BLK = 400


def _sc_body(x_hbm, src_hbm, dst_hbm,
             acc_out, deg_out,
             src_v, dst_v, rows_a, rows_b, deg_v,
             acc_sh, sem_a, sem_b):
    c = lax.axis_index("c")
    s = lax.axis_index("s")
    cpt = src_v.shape[0]

    zeros16 = jnp.zeros((LANES,), jnp.float32)
    ones16 = jnp.ones((LANES,), jnp.float32)

    def _zrow(i, _):
        for k in range(D // LANES):
            rows_a[i, pl.ds(k * LANES, LANES)] = zeros16
        return 0
    lax.fori_loop(0, CHUNK, _zrow, 0)

    def _zdeg(i, _):
        deg_v[pl.ds(i * LANES, LANES)] = zeros16
        return 0
    lax.fori_loop(0, R_HALF // LANES, _zdeg, 0)

    for i in range(ZPT // CHUNK):
        pltpu.sync_copy(rows_a, acc_sh.at[pl.ds(s * ZPT + i * CHUNK, CHUNK)])

    pltpu.sync_copy(src_hbm.at[s], src_v)
    pltpu.sync_copy(dst_hbm.at[c, s], dst_v)
    plsc.subcore_barrier()

    def _deg_update(j):
        for k in range(CHUNK // LANES):
            idx = dst_v[j, pl.ds(k * LANES, LANES)]
            plsc.addupdate_scatter(deg_v, [idx], ones16)

    pltpu.async_copy(x_hbm.at[src_v.at[0]], rows_a, sem_a)

    def _pair(j0, issue_next):
        j1 = j0 + 1
        pltpu.make_async_copy(x_hbm.at[src_v.at[j0]], rows_a, sem_a).wait()
        pltpu.async_copy(x_hbm.at[src_v.at[j1]], rows_b, sem_b)
        _deg_update(j0)
        pltpu.sync_copy(rows_a, acc_sh.at[dst_v.at[j0]], add=True)
        pltpu.make_async_copy(x_hbm.at[src_v.at[j1]], rows_b, sem_b).wait()
        if issue_next:
            pltpu.async_copy(x_hbm.at[src_v.at[j1 + 1]], rows_a, sem_a)
        _deg_update(j1)
        pltpu.sync_copy(rows_b, acc_sh.at[dst_v.at[j1]], add=True)

    def _step(t, _):
        _pair(2 * t, True)
        return 0

    lax.fori_loop(0, cpt // 2 - 1, _step, 0)
    _pair(cpt - 2, False)

    pltpu.sync_copy(deg_v, deg_out.at[c * NS + s])
    plsc.subcore_barrier()

    off = 0
    while off < OPT:
        n = min(CHUNK, OPT - off)
        pltpu.sync_copy(acc_sh.at[pl.ds(s * OPT + off, n)],
                        rows_a.at[pl.ds(0, n)])
        pltpu.sync_copy(rows_a.at[pl.ds(0, n)],
                        acc_out.at[pl.ds(c * HALF + s * OPT + off, n)])
        off += n


def _sc_aggregate(x_entity, src3, dst4):
    cpt = src3.shape[1]
    mesh = plsc.VectorSubcoreMesh(core_axis_name="c", subcore_axis_name="s")
    return pl.kernel(
        _sc_body,
        out_type=(
            jax.ShapeDtypeStruct((R_ACC, D), jnp.float32),
            jax.ShapeDtypeStruct((NW, R_HALF), jnp.float32),
        ),
        mesh=mesh,
        compiler_params=pltpu.CompilerParams(needs_layout_passes=False),
        scratch_types=[
            pltpu.VMEM((cpt, CHUNK), jnp.int32),
            pltpu.VMEM((cpt, CHUNK), jnp.int32),
            pltpu.VMEM((CHUNK, D), jnp.float32),
            pltpu.VMEM((CHUNK, D), jnp.float32),
            pltpu.VMEM((R_HALF,), jnp.float32),
            pltpu.VMEM_SHARED((R_HALF, D), jnp.float32),
            pltpu.SemaphoreType.DMA,
            pltpu.SemaphoreType.DMA,
        ],
    )(x_entity, src3, dst4)


def _ent_body(x_ref, acc_ref, deg_ref, wskip_ref, wee_ref, wtrans_ref,
              bskip_ref, bee_ref, btrans_ref, out_ref):
    agg = acc_ref[...]
    d = jnp.sum(deg_ref[0], axis=0)
    d = jnp.maximum(d, 1.0)
    x = x_ref[...]
    h = jnp.dot(x, wskip_ref[...], preferred_element_type=jnp.float32)
    h = h + jnp.dot(agg / d[:, None], wee_ref[...],
                    preferred_element_type=jnp.float32)
    h = h + bskip_ref[...] + bee_ref[...]
    h = jnp.where(h >= 0, h, 0.01 * h)
    out_ref[...] = jnp.dot(h, wtrans_ref[...],
                           preferred_element_type=jnp.float32) + btrans_ref[...]


def _ent_path(x_entity, acc, deg4, w_skip, w_ee, w_trans, b_skip, b_ee, b_trans):
    blk = ENT_BLK
    grid = N_ENT // blk
    wspec = pl.BlockSpec((D, D), lambda i: (0, 0))
    bspec = pl.BlockSpec((1, D), lambda i: (0, 0))
    return pl.pallas_call(
        _ent_body,
        grid=(grid,),
        in_specs=[
            pl.BlockSpec((blk, D), lambda i: (i, 0)),
            pl.BlockSpec((blk, D), lambda i: (i, 0)),
            pl.BlockSpec((1, NS, blk), lambda i: (i, 0, 0)),
            wspec, wspec, wspec, bspec, bspec, bspec,
        ],
        out_specs=pl.BlockSpec((blk, D), lambda i: (i, 0)),
        out_shape=jax.ShapeDtypeStruct((N_ENT, D), jnp.float32),
    )(x_entity, acc, deg4, w_skip, w_ee, w_trans,
      b_skip.reshape(1, D), b_ee.reshape(1, D), b_trans.reshape(1, D))


def _snap_body(xs_ref, acc_ref, deg_ref, ss_src_ref, ss_dst_ref,
               wskip_ref, wes_ref, wss_ref, wtrans_ref,
               bskip_ref, bes_ref, bss_ref, btrans_ref, out_ref):
    m = 16
    aggs = acc_ref[...]
    rowid = lax.broadcasted_iota(jnp.int32, (m, D), 0)
    aggs = jnp.where(rowid < N_SNAP, aggs, 0.0)
    ds_ = jnp.sum(deg_ref[...], axis=0)
    ds_ = jnp.maximum(ds_, 1.0)
    conv_es = jnp.dot(aggs / ds_[:, None], wes_ref[...],
                      preferred_element_type=jnp.float32) + bes_ref[...]
    xs = xs_ref[...]
    h0 = jnp.dot(xs, wskip_ref[...],
                 preferred_element_type=jnp.float32) + bskip_ref[...]
    h0 = h0 + conv_es[:N_SNAP]

    colid = lax.broadcasted_iota(jnp.int32, (m, D), 1)
    a = jnp.zeros((m, D), jnp.float32)
    for e in range(ss_src_ref.shape[0]):
        se = ss_src_ref[e]
        de = ss_dst_ref[e]
        a = a + jnp.where((rowid == de) & (colid == se), 1.0, 0.0)
    h0p = jnp.concatenate([h0, jnp.zeros((D - N_SNAP, D), jnp.float32)], axis=0)
    aggss = jnp.dot(a, h0p, preferred_element_type=jnp.float32)
    degss = jnp.maximum(jnp.sum(a, axis=1), 1.0)
    hs = jnp.dot(aggss / degss[:, None], wss_ref[...],
                 preferred_element_type=jnp.float32) + bss_ref[...]
    hs = jnp.where(hs >= 0, hs, 0.01 * hs)
    res = jnp.dot(hs, wtrans_ref[...],
                  preferred_element_type=jnp.float32) + btrans_ref[...]
    out_ref[...] = res[:N_SNAP]


def _snap_path(x_snapshot, acc_es, deg_es, ss_src, ss_dst,
               w_skip, w_es, w_ss, w_trans, b_skip, b_es, b_ss, b_trans):
    wspec = pl.BlockSpec((D, D), lambda: (0, 0))
    bspec = pl.BlockSpec((1, D), lambda: (0, 0))
    sspec = pl.BlockSpec(memory_space=pltpu.SMEM)
    return pl.pallas_call(
        _snap_body,
        in_specs=[
            pl.BlockSpec((N_SNAP, D), lambda: (0, 0)),
            pl.BlockSpec((16, D), lambda: (0, 0)),
            pl.BlockSpec((NS, 16), lambda: (0, 0)),
            sspec, sspec,
            wspec, wspec, wspec, wspec, bspec, bspec, bspec, bspec,
        ],
        out_specs=pl.BlockSpec((N_SNAP, D), lambda: (0, 0)),
        out_shape=jax.ShapeDtypeStruct((N_SNAP, D), jnp.float32),
    )(x_snapshot, acc_es, deg_es, ss_src, ss_dst,
      w_skip, w_es, w_ss, w_trans,
      b_skip.reshape(1, D), b_es.reshape(1, D), b_ss.reshape(1, D),
      b_trans.reshape(1, D))


def kernel(x_entity, x_snapshot, ee_src, ee_dst, es_src, es_dst, ss_src, ss_dst,
           W_ee, b_ee, W_es, b_es, W_ss, b_ss,
           W_skip_ent, b_skip_ent, W_skip_snap, b_skip_snap,
           W_trans_ent, b_trans_ent, W_trans_snap, b_trans_snap):
    e_total = ee_src.shape[0] + es_src.shape[0]
    e_pad = -(-e_total // (NS * CHUNK * 2)) * (NS * CHUNK * 2)
    npad = e_pad - e_total
    src3 = jnp.concatenate(
        [ee_src, es_src, jnp.zeros((npad,), jnp.int32)]).reshape(NS, -1, CHUNK)
    dstg = jnp.concatenate(
        [ee_dst, es_dst + N_ENT, jnp.full((npad,), DUMMY, jnp.int32)])
    oob_row = HALF + (jnp.arange(e_pad, dtype=jnp.int32) & (R_HALF - HALF - 1))
    d0 = jnp.where(dstg < HALF, dstg, oob_row)
    d1t = dstg - HALF
    d1 = jnp.where((d1t >= 0) & (d1t < HALF), d1t, oob_row)
    dst4 = jnp.stack([d0, d1]).reshape(NC, NS, -1, CHUNK)

    acc, deg = _sc_aggregate(x_entity, src3, dst4)
    dglob = (deg.reshape(NC, NS, R_HALF)[:, :, :HALF]
             .transpose(1, 0, 2).reshape(NS, R_ACC))

    deg4 = (dglob[:, :N_ENT].reshape(NS, N_ENT // ENT_BLK, ENT_BLK)
            .transpose(1, 0, 2))
    h_ent = _ent_path(x_entity, acc, deg4, W_skip_ent, W_ee, W_trans_ent,
                      b_skip_ent, b_ee, b_trans_ent)

    acc_es = acc[N_ENT:N_ENT + 16]
    deg_es = dglob[:, N_ENT:N_ENT + 16]
    h_snap = _snap_path(x_snapshot, acc_es, deg_es, ss_src, ss_dst,
                        W_skip_snap, W_es, W_ss, W_trans_snap,
                        b_skip_snap, b_es, b_ss, b_trans_snap)
    return (h_ent, h_snap)


# speedup vs baseline: 5.2066x; 1.1260x over previous
import jax
import jax.numpy as jnp
from jax import lax
from jax.experimental import pallas as pl
from jax.experimental.pallas import tpu as pltpu
from jax.experimental.pallas import tpu_sc as plsc

N_ENT = 10000
N_SNAP = 10
D = 128

NC = 2
NS = 16
NW = NC * NS
LANES = 16
CHUNK = 128
SEG = 27

HALF = 5120
R_HALF = 6144
R_ACC = 2 * HALF
DUMMY = N_ENT + N_SNAP
ZPT = R_HALF // NS
OPT = HALF // NS
ENT_BLK = 400


def _sc_body(x_hbm, src_hbm, dst_hbm,
             acc_out, deg_out,
             src_i0, src_i1, dst_i0, dst_i1,
             rows_a, rows_b, rows_c, deg_v,
             acc_sh, sg0, sg1, sg2, ss0, ss1, ss2, si):
    c = lax.axis_index("c")
    s = lax.axis_index("s")
    nseg = src_hbm.shape[1]
    rows = (rows_a, rows_b, rows_c)
    sg = (sg0, sg1, sg2)
    ss = (ss0, ss1, ss2)

    zeros16 = jnp.zeros((LANES,), jnp.float32)
    ones16 = jnp.ones((LANES,), jnp.float32)

    def _zrow(i, _):
        for k in range(D // LANES):
            rows_a[i, pl.ds(k * LANES, LANES)] = zeros16
        return 0
    lax.fori_loop(0, CHUNK, _zrow, 0)

    def _zdeg(i, _):
        deg_v[pl.ds(i * LANES, LANES)] = zeros16
        return 0
    lax.fori_loop(0, R_HALF // LANES, _zdeg, 0)

    for i in range(ZPT // CHUNK):
        pltpu.sync_copy(rows_a, acc_sh.at[pl.ds(s * ZPT + i * CHUNK, CHUNK)])

    pltpu.sync_copy(src_hbm.at[s, 0], src_i0)
    pltpu.sync_copy(dst_hbm.at[c, s, 0], dst_i0)
    pltpu.sync_copy(src_hbm.at[s, 1], src_i1)
    pltpu.sync_copy(dst_hbm.at[c, s, 1], dst_i1)
    plsc.subcore_barrier()

    def _deg_update(dref, lj):
        for k in range(CHUNK // LANES):
            idx = dref[lj, pl.ds(k * LANES, LANES)]
            plsc.addupdate_scatter(deg_v, [idx], ones16)

    def _slot(lj, k, sref, dref, wait_s, gref=None, glj=None):
        k2 = (k + 2) % 3
        pltpu.make_async_copy(x_hbm.at[sref.at[lj]], rows[k], sg[k]).wait()
        _deg_update(dref, lj)
        pltpu.async_copy(rows[k], acc_sh.at[dref.at[lj]], ss[k], add=True)
        if wait_s:
            pltpu.make_async_copy(rows[k2], acc_sh.at[dref.at[lj]],
                                  ss[k2]).wait()
        if gref is not None:
            pltpu.async_copy(x_hbm.at[gref.at[glj]], rows[k2], sg[k2])

    pltpu.async_copy(x_hbm.at[src_i0.at[0]], rows_a, sg0)
    pltpu.async_copy(x_hbm.at[src_i0.at[1]], rows_b, sg1)

    for g in range(nseg):
        if g % 2 == 0:
            sref, dref, srefn, drefn = src_i0, dst_i0, src_i1, dst_i1
        else:
            sref, dref, srefn, drefn = src_i1, dst_i1, src_i0, dst_i0
        last = g == nseg - 1
        _slot(0, 0, sref, dref, wait_s=(g > 0), gref=sref, glj=2)
        if 0 < g < nseg - 1:
            pltpu.async_copy(src_hbm.at[s, g + 1], srefn, si)
            pltpu.async_copy(dst_hbm.at[c, s, g + 1], drefn, si)
        _slot(1, 1, sref, dref, True, sref, 3)
        _slot(2, 2, sref, dref, True, sref, 4)

        def _mid(t, _):
            l0 = 3 * t
            _slot(l0, 0, sref, dref, True, sref, l0 + 2)
            _slot(l0 + 1, 1, sref, dref, True, sref, l0 + 3)
            _slot(l0 + 2, 2, sref, dref, True, sref, l0 + 4)
            return 0

        lax.fori_loop(1, SEG // 3 - 1, _mid, 0)
        _slot(SEG - 3, 0, sref, dref, True, sref, SEG - 1)
        if not last:
            if g > 0:
                pltpu.make_async_copy(src_hbm.at[s, g + 1], srefn, si).wait()
                pltpu.make_async_copy(dst_hbm.at[c, s, g + 1], drefn,
                                      si).wait()
            _slot(SEG - 2, 1, sref, dref, True, srefn, 0)
            _slot(SEG - 1, 2, sref, dref, True, srefn, 1)
        else:
            _slot(SEG - 2, 1, sref, dref, True)
            _slot(SEG - 1, 2, sref, dref, True)

    lastd = dst_i0 if (nseg - 1) % 2 == 0 else dst_i1
    pltpu.make_async_copy(rows[2], acc_sh.at[lastd.at[SEG - 1]], ss[2]).wait()

    pltpu.sync_copy(deg_v, deg_out.at[c * NS + s])
    plsc.subcore_barrier()

    off = 0
    while off < OPT:
        n = min(CHUNK, OPT - off)
        pltpu.sync_copy(acc_sh.at[pl.ds(s * OPT + off, n)],
                        rows_a.at[pl.ds(0, n)])
        pltpu.sync_copy(rows_a.at[pl.ds(0, n)],
                        acc_out.at[pl.ds(c * HALF + s * OPT + off, n)])
        off += n


def _sc_aggregate(x_entity, src3, dst4):
    cpt = src3.shape[1]
    mesh = plsc.VectorSubcoreMesh(core_axis_name="c", subcore_axis_name="s")
    return pl.kernel(
        _sc_body,
        out_type=(
            jax.ShapeDtypeStruct((R_ACC, D), jnp.float32),
            jax.ShapeDtypeStruct((NW, R_HALF), jnp.float32),
        ),
        mesh=mesh,
        compiler_params=pltpu.CompilerParams(needs_layout_passes=False),
        scratch_types=[
            pltpu.VMEM((SEG, CHUNK), jnp.int32),
            pltpu.VMEM((SEG, CHUNK), jnp.int32),
            pltpu.VMEM((SEG, CHUNK), jnp.int32),
            pltpu.VMEM((SEG, CHUNK), jnp.int32),
            pltpu.VMEM((CHUNK, D), jnp.float32),
            pltpu.VMEM((CHUNK, D), jnp.float32),
            pltpu.VMEM((CHUNK, D), jnp.float32),
            pltpu.VMEM((R_HALF,), jnp.float32),
            pltpu.VMEM_SHARED((R_HALF, D), jnp.float32),
            pltpu.SemaphoreType.DMA,
            pltpu.SemaphoreType.DMA,
            pltpu.SemaphoreType.DMA,
            pltpu.SemaphoreType.DMA,
            pltpu.SemaphoreType.DMA,
            pltpu.SemaphoreType.DMA,
            pltpu.SemaphoreType.DMA,
        ],
    )(x_entity, src3, dst4)


def _ent_body(x_ref, acc_ref, deg_ref, wskip_ref, wee_ref, wtrans_ref,
              bskip_ref, bee_ref, btrans_ref, out_ref):
    agg = acc_ref[...]
    d = jnp.sum(deg_ref[0], axis=0)
    d = jnp.maximum(d, 1.0)
    x = x_ref[...]
    h = jnp.dot(x, wskip_ref[...], preferred_element_type=jnp.float32)
    h = h + jnp.dot(agg / d[:, None], wee_ref[...],
                    preferred_element_type=jnp.float32)
    h = h + bskip_ref[...] + bee_ref[...]
    h = jnp.where(h >= 0, h, 0.01 * h)
    out_ref[...] = jnp.dot(h, wtrans_ref[...],
                           preferred_element_type=jnp.float32) + btrans_ref[...]


def _ent_path(x_entity, acc, deg4, w_skip, w_ee, w_trans, b_skip, b_ee, b_trans):
    blk = ENT_BLK
    grid = N_ENT // blk
    wspec = pl.BlockSpec((D, D), lambda i: (0, 0))
    bspec = pl.BlockSpec((1, D), lambda i: (0, 0))
    return pl.pallas_call(
        _ent_body,
        grid=(grid,),
        in_specs=[
            pl.BlockSpec((blk, D), lambda i: (i, 0)),
            pl.BlockSpec((blk, D), lambda i: (i, 0)),
            pl.BlockSpec((1, NS, blk), lambda i: (i, 0, 0)),
            wspec, wspec, wspec, bspec, bspec, bspec,
        ],
        out_specs=pl.BlockSpec((blk, D), lambda i: (i, 0)),
        out_shape=jax.ShapeDtypeStruct((N_ENT, D), jnp.float32),
    )(x_entity, acc, deg4, w_skip, w_ee, w_trans,
      b_skip.reshape(1, D), b_ee.reshape(1, D), b_trans.reshape(1, D))


def _snap_body(xs_ref, acc_ref, deg_ref, ss_src_ref, ss_dst_ref,
               wskip_ref, wes_ref, wss_ref, wtrans_ref,
               bskip_ref, bes_ref, bss_ref, btrans_ref, out_ref):
    m = 16
    aggs = acc_ref[...]
    rowid = lax.broadcasted_iota(jnp.int32, (m, D), 0)
    aggs = jnp.where(rowid < N_SNAP, aggs, 0.0)
    ds_ = jnp.sum(deg_ref[...], axis=0)
    ds_ = jnp.maximum(ds_, 1.0)
    conv_es = jnp.dot(aggs / ds_[:, None], wes_ref[...],
                      preferred_element_type=jnp.float32) + bes_ref[...]
    xs = xs_ref[...]
    h0 = jnp.dot(xs, wskip_ref[...],
                 preferred_element_type=jnp.float32) + bskip_ref[...]
    h0 = h0 + conv_es[:N_SNAP]

    colid = lax.broadcasted_iota(jnp.int32, (m, D), 1)
    a = jnp.zeros((m, D), jnp.float32)
    for e in range(ss_src_ref.shape[0]):
        se = ss_src_ref[e]
        de = ss_dst_ref[e]
        a = a + jnp.where((rowid == de) & (colid == se), 1.0, 0.0)
    h0p = jnp.concatenate([h0, jnp.zeros((D - N_SNAP, D), jnp.float32)], axis=0)
    aggss = jnp.dot(a, h0p, preferred_element_type=jnp.float32)
    degss = jnp.maximum(jnp.sum(a, axis=1), 1.0)
    hs = jnp.dot(aggss / degss[:, None], wss_ref[...],
                 preferred_element_type=jnp.float32) + bss_ref[...]
    hs = jnp.where(hs >= 0, hs, 0.01 * hs)
    res = jnp.dot(hs, wtrans_ref[...],
                  preferred_element_type=jnp.float32) + btrans_ref[...]
    out_ref[...] = res[:N_SNAP]


def _snap_path(x_snapshot, acc_es, deg_es, ss_src, ss_dst,
               w_skip, w_es, w_ss, w_trans, b_skip, b_es, b_ss, b_trans):
    wspec = pl.BlockSpec((D, D), lambda: (0, 0))
    bspec = pl.BlockSpec((1, D), lambda: (0, 0))
    sspec = pl.BlockSpec(memory_space=pltpu.SMEM)
    return pl.pallas_call(
        _snap_body,
        in_specs=[
            pl.BlockSpec((N_SNAP, D), lambda: (0, 0)),
            pl.BlockSpec((16, D), lambda: (0, 0)),
            pl.BlockSpec((NS, 16), lambda: (0, 0)),
            sspec, sspec,
            wspec, wspec, wspec, wspec, bspec, bspec, bspec, bspec,
        ],
        out_specs=pl.BlockSpec((N_SNAP, D), lambda: (0, 0)),
        out_shape=jax.ShapeDtypeStruct((N_SNAP, D), jnp.float32),
    )(x_snapshot, acc_es, deg_es, ss_src, ss_dst,
      w_skip, w_es, w_ss, w_trans,
      b_skip.reshape(1, D), b_es.reshape(1, D), b_ss.reshape(1, D),
      b_trans.reshape(1, D))


def kernel(x_entity, x_snapshot, ee_src, ee_dst, es_src, es_dst, ss_src, ss_dst,
           W_ee, b_ee, W_es, b_es, W_ss, b_ss,
           W_skip_ent, b_skip_ent, W_skip_snap, b_skip_snap,
           W_trans_ent, b_trans_ent, W_trans_snap, b_trans_snap):
    e_total = ee_src.shape[0] + es_src.shape[0]
    e_pad = -(-e_total // (NS * CHUNK * SEG)) * (NS * CHUNK * SEG)
    npad = e_pad - e_total
    src3 = jnp.concatenate(
        [ee_src, es_src, jnp.zeros((npad,), jnp.int32)]).reshape(NS, -1, SEG, CHUNK)
    dstg = jnp.concatenate(
        [ee_dst, es_dst + N_ENT, jnp.full((npad,), DUMMY, jnp.int32)])
    oob_row = HALF + (jnp.arange(e_pad, dtype=jnp.int32) & (R_HALF - HALF - 1))
    d0 = jnp.where(dstg < HALF, dstg, oob_row)
    d1t = dstg - HALF
    d1 = jnp.where((d1t >= 0) & (d1t < HALF), d1t, oob_row)
    dst4 = jnp.stack([d0, d1]).reshape(NC, NS, -1, SEG, CHUNK)

    acc, deg = _sc_aggregate(x_entity, src3, dst4)
    dglob = (deg.reshape(NC, NS, R_HALF)[:, :, :HALF]
             .transpose(1, 0, 2).reshape(NS, R_ACC))

    deg4 = (dglob[:, :N_ENT].reshape(NS, N_ENT // ENT_BLK, ENT_BLK)
            .transpose(1, 0, 2))
    h_ent = _ent_path(x_entity, acc, deg4, W_skip_ent, W_ee, W_trans_ent,
                      b_skip_ent, b_ee, b_trans_ent)

    acc_es = acc[N_ENT:N_ENT + 16]
    deg_es = dglob[:, N_ENT:N_ENT + 16]
    h_snap = _snap_path(x_snapshot, acc_es, deg_es, ss_src, ss_dst,
                        W_skip_snap, W_es, W_ss, W_trans_snap,
                        b_skip_snap, b_es, b_ss, b_trans_snap)
    return (h_ent, h_snap)


# speedup vs baseline: 7.5693x; 1.4538x over previous
import jax
import jax.numpy as jnp
from jax import lax
from jax.experimental import pallas as pl
from jax.experimental.pallas import tpu as pltpu
from jax.experimental.pallas import tpu_sc as plsc

N_ENT = 10000
N_SNAP = 10
D = 128
DH = D // 2

NC = 2
NS = 16
NW = NC * NS
LANES = 16
CHUNK = 128
SEG = 27

R_ACC = 10240
DUMMY = N_ENT + N_SNAP
ZPT = R_ACC // NS
ENT_BLK = 400


def _sc_body(x_hbm, src_hbm, dst_hbm,
             acc_out, deg_out,
             src_i0, src_i1, dst_i0, dst_i1,
             rows_a, rows_b, rows_c, deg_v,
             acc_sh, sg0, sg1, sg2, ss0, ss1, ss2, si):
    c = lax.axis_index("c")
    s = lax.axis_index("s")
    nseg = dst_hbm.shape[1]
    rows = (rows_a, rows_b, rows_c)
    sg = (sg0, sg1, sg2)
    ss = (ss0, ss1, ss2)

    zeros16 = jnp.zeros((LANES,), jnp.float32)
    ones16 = jnp.ones((LANES,), jnp.float32)

    def _zrow(i, _):
        for k in range(DH // LANES):
            rows_a[i, pl.ds(k * LANES, LANES)] = zeros16
        return 0
    lax.fori_loop(0, CHUNK, _zrow, 0)

    def _zdeg(i, _):
        deg_v[pl.ds(i * LANES, LANES)] = zeros16
        return 0
    lax.fori_loop(0, R_ACC // LANES, _zdeg, 0)

    for i in range(ZPT // CHUNK):
        pltpu.sync_copy(rows_a, acc_sh.at[pl.ds(s * ZPT + i * CHUNK, CHUNK)])

    pltpu.sync_copy(src_hbm.at[c, s, 0], src_i0)
    pltpu.sync_copy(dst_hbm.at[s, 0], dst_i0)
    pltpu.sync_copy(src_hbm.at[c, s, 1], src_i1)
    pltpu.sync_copy(dst_hbm.at[s, 1], dst_i1)
    plsc.subcore_barrier()

    def _deg_update(dref, lj):
        for k in range(CHUNK // LANES):
            idx = dref[lj, pl.ds(k * LANES, LANES)]
            plsc.addupdate_scatter(deg_v, [idx], ones16)

    def _slot(lj, k, sref, dref, wait_s, gref=None, glj=None):
        k2 = (k + 2) % 3
        pltpu.make_async_copy(x_hbm.at[sref.at[lj]], rows[k], sg[k]).wait()
        _deg_update(dref, lj)
        pltpu.async_copy(rows[k], acc_sh.at[dref.at[lj]], ss[k], add=True)
        if wait_s:
            pltpu.make_async_copy(rows[k2], acc_sh.at[dref.at[lj]],
                                  ss[k2]).wait()
        if gref is not None:
            pltpu.async_copy(x_hbm.at[gref.at[glj]], rows[k2], sg[k2])

    pltpu.async_copy(x_hbm.at[src_i0.at[0]], rows_a, sg0)
    pltpu.async_copy(x_hbm.at[src_i0.at[1]], rows_b, sg1)

    for g in range(nseg):
        if g % 2 == 0:
            sref, dref, srefn, drefn = src_i0, dst_i0, src_i1, dst_i1
        else:
            sref, dref, srefn, drefn = src_i1, dst_i1, src_i0, dst_i0
        last = g == nseg - 1
        _slot(0, 0, sref, dref, wait_s=(g > 0), gref=sref, glj=2)
        if 0 < g < nseg - 1:
            pltpu.async_copy(src_hbm.at[c, s, g + 1], srefn, si)
            pltpu.async_copy(dst_hbm.at[s, g + 1], drefn, si)
        _slot(1, 1, sref, dref, True, sref, 3)
        _slot(2, 2, sref, dref, True, sref, 4)

        def _mid(t, _):
            l0 = 3 * t
            _slot(l0, 0, sref, dref, True, sref, l0 + 2)
            _slot(l0 + 1, 1, sref, dref, True, sref, l0 + 3)
            _slot(l0 + 2, 2, sref, dref, True, sref, l0 + 4)
            return 0

        lax.fori_loop(1, SEG // 3 - 1, _mid, 0)
        _slot(SEG - 3, 0, sref, dref, True, sref, SEG - 1)
        if not last:
            if g > 0:
                pltpu.make_async_copy(src_hbm.at[c, s, g + 1], srefn,
                                      si).wait()
                pltpu.make_async_copy(dst_hbm.at[s, g + 1], drefn, si).wait()
            _slot(SEG - 2, 1, sref, dref, True, srefn, 0)
            _slot(SEG - 1, 2, sref, dref, True, srefn, 1)
        else:
            _slot(SEG - 2, 1, sref, dref, True)
            _slot(SEG - 1, 2, sref, dref, True)

    lastd = dst_i0 if (nseg - 1) % 2 == 0 else dst_i1
    pltpu.make_async_copy(rows[2], acc_sh.at[lastd.at[SEG - 1]], ss[2]).wait()

    pltpu.sync_copy(deg_v, deg_out.at[c * NS + s])
    plsc.subcore_barrier()

    for i in range(ZPT // CHUNK):
        r0 = s * ZPT + i * CHUNK
        pltpu.sync_copy(acc_sh.at[pl.ds(r0, CHUNK)], rows_a)
        pltpu.sync_copy(rows_a, acc_out.at[c, pl.ds(r0, CHUNK)])


def _sc_aggregate(x_halves, src5, dst4):
    mesh = plsc.VectorSubcoreMesh(core_axis_name="c", subcore_axis_name="s")
    return pl.kernel(
        _sc_body,
        out_type=(
            jax.ShapeDtypeStruct((NC, R_ACC, DH), jnp.float32),
            jax.ShapeDtypeStruct((NW, R_ACC), jnp.float32),
        ),
        mesh=mesh,
        compiler_params=pltpu.CompilerParams(needs_layout_passes=False,
                                             use_tc_tiling_on_sc=False),
        scratch_types=[
            pltpu.VMEM((SEG, CHUNK), jnp.int32),
            pltpu.VMEM((SEG, CHUNK), jnp.int32),
            pltpu.VMEM((SEG, CHUNK), jnp.int32),
            pltpu.VMEM((SEG, CHUNK), jnp.int32),
            pltpu.VMEM((CHUNK, DH), jnp.float32),
            pltpu.VMEM((CHUNK, DH), jnp.float32),
            pltpu.VMEM((CHUNK, DH), jnp.float32),
            pltpu.VMEM((R_ACC,), jnp.float32),
            pltpu.VMEM_SHARED((R_ACC, DH), jnp.float32),
            pltpu.SemaphoreType.DMA,
            pltpu.SemaphoreType.DMA,
            pltpu.SemaphoreType.DMA,
            pltpu.SemaphoreType.DMA,
            pltpu.SemaphoreType.DMA,
            pltpu.SemaphoreType.DMA,
            pltpu.SemaphoreType.DMA,
        ],
    )(x_halves, src5, dst4)


def _ent_body(x_ref, acc_ref, deg_ref, wskip_ref, wee_ref, wtrans_ref,
              bskip_ref, bee_ref, btrans_ref, out_ref):
    d = jnp.sum(deg_ref[0], axis=0) * 0.5
    d = jnp.maximum(d, 1.0)
    r = 1.0 / d[:, None]
    x = x_ref[...]
    h = jnp.dot(x, wskip_ref[...], preferred_element_type=jnp.float32)
    h = h + jnp.dot(acc_ref[0] * r, wee_ref[0, :DH, :],
                    preferred_element_type=jnp.float32)
    h = h + jnp.dot(acc_ref[1] * r, wee_ref[0, DH:, :],
                    preferred_element_type=jnp.float32)
    h = h + bskip_ref[...] + bee_ref[...]
    h = jnp.where(h >= 0, h, 0.01 * h)
    out_ref[...] = jnp.dot(h, wtrans_ref[...],
                           preferred_element_type=jnp.float32) + btrans_ref[...]


def _ent_path(x_entity, acc, deg4, w_skip, w_ee, w_trans, b_skip, b_ee, b_trans):
    blk = ENT_BLK
    grid = N_ENT // blk
    wspec = pl.BlockSpec((D, D), lambda i: (0, 0))
    bspec = pl.BlockSpec((1, D), lambda i: (0, 0))
    return pl.pallas_call(
        _ent_body,
        grid=(grid,),
        in_specs=[
            pl.BlockSpec((blk, D), lambda i: (i, 0)),
            pl.BlockSpec((NC, blk, DH), lambda i: (0, i, 0)),
            pl.BlockSpec((1, NW, blk), lambda i: (i, 0, 0)),
            wspec, pl.BlockSpec((1, D, D), lambda i: (0, 0, 0)), wspec,
            bspec, bspec, bspec,
        ],
        out_specs=pl.BlockSpec((blk, D), lambda i: (i, 0)),
        out_shape=jax.ShapeDtypeStruct((N_ENT, D), jnp.float32),
    )(x_entity, acc, deg4, w_skip, w_ee.reshape(1, D, D), w_trans,
      b_skip.reshape(1, D), b_ee.reshape(1, D), b_trans.reshape(1, D))


def _snap_body(xs_ref, acc_ref, deg_ref, ss_src_ref, ss_dst_ref,
               wskip_ref, wes_ref, wss_ref, wtrans_ref,
               bskip_ref, bes_ref, bss_ref, btrans_ref, out_ref):
    m = 16
    rowid = lax.broadcasted_iota(jnp.int32, (m, D), 0)
    rowidh = lax.broadcasted_iota(jnp.int32, (m, DH), 0)
    ds_ = jnp.sum(deg_ref[...], axis=0) * 0.5
    ds_ = jnp.maximum(ds_, 1.0)
    r = 1.0 / ds_[:, None]
    aggl = jnp.where(rowidh < N_SNAP, acc_ref[0], 0.0) * r
    aggr = jnp.where(rowidh < N_SNAP, acc_ref[1], 0.0) * r
    conv_es = (jnp.dot(aggl, wes_ref[0, :DH, :],
                       preferred_element_type=jnp.float32)
               + jnp.dot(aggr, wes_ref[0, DH:, :],
                         preferred_element_type=jnp.float32)) + bes_ref[...]
    xs = xs_ref[...]
    h0 = jnp.dot(xs, wskip_ref[...],
                 preferred_element_type=jnp.float32) + bskip_ref[...]
    h0 = h0 + conv_es[:N_SNAP]

    colid = lax.broadcasted_iota(jnp.int32, (m, D), 1)
    a = jnp.zeros((m, D), jnp.float32)
    for e in range(ss_src_ref.shape[0]):
        se = ss_src_ref[e]
        de = ss_dst_ref[e]
        a = a + jnp.where((rowid == de) & (colid == se), 1.0, 0.0)
    h0p = jnp.concatenate([h0, jnp.zeros((D - N_SNAP, D), jnp.float32)], axis=0)
    aggss = jnp.dot(a, h0p, preferred_element_type=jnp.float32)
    degss = jnp.maximum(jnp.sum(a, axis=1), 1.0)
    hs = jnp.dot(aggss / degss[:, None], wss_ref[...],
                 preferred_element_type=jnp.float32) + bss_ref[...]
    hs = jnp.where(hs >= 0, hs, 0.01 * hs)
    res = jnp.dot(hs, wtrans_ref[...],
                  preferred_element_type=jnp.float32) + btrans_ref[...]
    out_ref[...] = res[:N_SNAP]


def _snap_path(x_snapshot, acc_es, deg_es, ss_src, ss_dst,
               w_skip, w_es, w_ss, w_trans, b_skip, b_es, b_ss, b_trans):
    wspec = pl.BlockSpec((D, D), lambda: (0, 0))
    bspec = pl.BlockSpec((1, D), lambda: (0, 0))
    sspec = pl.BlockSpec(memory_space=pltpu.SMEM)
    return pl.pallas_call(
        _snap_body,
        in_specs=[
            pl.BlockSpec((N_SNAP, D), lambda: (0, 0)),
            pl.BlockSpec((NC, 16, DH), lambda: (0, 0, 0)),
            pl.BlockSpec((NW, 16), lambda: (0, 0)),
            sspec, sspec,
            wspec, pl.BlockSpec((1, D, D), lambda: (0, 0, 0)), wspec, wspec,
            bspec, bspec, bspec, bspec,
        ],
        out_specs=pl.BlockSpec((N_SNAP, D), lambda: (0, 0)),
        out_shape=jax.ShapeDtypeStruct((N_SNAP, D), jnp.float32),
    )(x_snapshot, acc_es, deg_es, ss_src, ss_dst,
      w_skip, w_es.reshape(1, D, D), w_ss, w_trans,
      b_skip.reshape(1, D), b_es.reshape(1, D), b_ss.reshape(1, D),
      b_trans.reshape(1, D))


def kernel(x_entity, x_snapshot, ee_src, ee_dst, es_src, es_dst, ss_src, ss_dst,
           W_ee, b_ee, W_es, b_es, W_ss, b_ss,
           W_skip_ent, b_skip_ent, W_skip_snap, b_skip_snap,
           W_trans_ent, b_trans_ent, W_trans_snap, b_trans_snap):
    e_total = ee_src.shape[0] + es_src.shape[0]
    e_pad = -(-e_total // (NS * CHUNK * SEG)) * (NS * CHUNK * SEG)
    npad = e_pad - e_total
    xh = jnp.concatenate([x_entity[:, :DH], x_entity[:, DH:]], axis=0)
    srcg = jnp.concatenate([ee_src, es_src, jnp.zeros((npad,), jnp.int32)])
    src5 = jnp.stack([srcg, srcg + N_ENT]).reshape(NC, NS, -1, SEG, CHUNK)
    padrows = DUMMY + (jnp.arange(npad, dtype=jnp.int32) & 127)
    dstg = jnp.concatenate([ee_dst, es_dst + N_ENT, padrows])
    dst4 = dstg.reshape(NS, -1, SEG, CHUNK)

    acc, deg = _sc_aggregate(xh, src5, dst4)

    deg4 = (deg[:, :N_ENT].reshape(NW, N_ENT // ENT_BLK, ENT_BLK)
            .transpose(1, 0, 2))
    h_ent = _ent_path(x_entity, acc, deg4, W_skip_ent, W_ee, W_trans_ent,
                      b_skip_ent, b_ee, b_trans_ent)

    acc_es = acc[:, N_ENT:N_ENT + 16, :]
    deg_es = deg[:, N_ENT:N_ENT + 16]
    h_snap = _snap_path(x_snapshot, acc_es, deg_es, ss_src, ss_dst,
                        W_skip_snap, W_es, W_ss, W_trans_snap,
                        b_skip_snap, b_es, b_ss, b_trans_snap)
    return (h_ent, h_snap)


# speedup vs baseline: 8.0951x; 1.0695x over previous
import jax
import jax.numpy as jnp
from jax import lax
from jax.experimental import pallas as pl
from jax.experimental.pallas import tpu as pltpu
from jax.experimental.pallas import tpu_sc as plsc

N_ENT = 10000
N_SNAP = 10
D = 128
DH = D // 2

NC = 2
NS = 16
NW = NC * NS
LANES = 16
CHUNK = 128
SEG = 27

R_ACC = 10240
DUMMY = N_ENT + N_SNAP
ZPT = R_ACC // NS
ENT_BLK = 2048


def _sc_body(x_hbm, src_hbm, dst_hbm,
             acc_out, deg_out,
             src_i0, src_i1, dst_i0, dst_i1,
             rows_a, rows_b, rows_c, deg_v,
             acc_sh, sg0, sg1, sg2, ss0, ss1, ss2, si):
    c = lax.axis_index("c")
    s = lax.axis_index("s")
    nseg = dst_hbm.shape[1]
    rows = (rows_a, rows_b, rows_c)
    sg = (sg0, sg1, sg2)
    ss = (ss0, ss1, ss2)

    zeros16 = jnp.zeros((LANES,), jnp.float32)
    ones16 = jnp.ones((LANES,), jnp.float32)

    def _zrow(i, _):
        for k in range(DH // LANES):
            rows_a[i, pl.ds(k * LANES, LANES)] = zeros16
        return 0
    lax.fori_loop(0, CHUNK, _zrow, 0)

    def _zdeg(i, _):
        deg_v[pl.ds(i * LANES, LANES)] = zeros16
        return 0
    lax.fori_loop(0, R_ACC // LANES, _zdeg, 0)

    for i in range(ZPT // CHUNK):
        pltpu.sync_copy(rows_a, acc_sh.at[pl.ds(s * ZPT + i * CHUNK, CHUNK)])

    pltpu.sync_copy(src_hbm.at[c, s, 0], src_i0)
    pltpu.sync_copy(dst_hbm.at[s, 0], dst_i0)
    pltpu.sync_copy(src_hbm.at[c, s, 1], src_i1)
    pltpu.sync_copy(dst_hbm.at[s, 1], dst_i1)
    plsc.subcore_barrier()

    def _deg_update(dref, lj):
        for k in range(CHUNK // LANES):
            idx = dref[lj, pl.ds(k * LANES, LANES)]
            plsc.addupdate_scatter(deg_v, [idx], ones16)

    def _slot(lj, k, sref, dref, wait_s, gref=None, glj=None):
        k2 = (k + 2) % 3
        pltpu.make_async_copy(x_hbm.at[sref.at[lj]], rows[k], sg[k]).wait()
        _deg_update(dref, lj)
        pltpu.async_copy(rows[k], acc_sh.at[dref.at[lj]], ss[k], add=True)
        if wait_s:
            pltpu.make_async_copy(rows[k2], acc_sh.at[dref.at[lj]],
                                  ss[k2]).wait()
        if gref is not None:
            pltpu.async_copy(x_hbm.at[gref.at[glj]], rows[k2], sg[k2])

    pltpu.async_copy(x_hbm.at[src_i0.at[0]], rows_a, sg0)
    pltpu.async_copy(x_hbm.at[src_i0.at[1]], rows_b, sg1)

    for g in range(nseg):
        if g % 2 == 0:
            sref, dref, srefn, drefn = src_i0, dst_i0, src_i1, dst_i1
        else:
            sref, dref, srefn, drefn = src_i1, dst_i1, src_i0, dst_i0
        last = g == nseg - 1
        _slot(0, 0, sref, dref, wait_s=(g > 0), gref=sref, glj=2)
        if 0 < g < nseg - 1:
            pltpu.async_copy(src_hbm.at[c, s, g + 1], srefn, si)
            pltpu.async_copy(dst_hbm.at[s, g + 1], drefn, si)
        _slot(1, 1, sref, dref, True, sref, 3)
        _slot(2, 2, sref, dref, True, sref, 4)

        def _mid(t, _):
            l0 = 3 * t
            _slot(l0, 0, sref, dref, True, sref, l0 + 2)
            _slot(l0 + 1, 1, sref, dref, True, sref, l0 + 3)
            _slot(l0 + 2, 2, sref, dref, True, sref, l0 + 4)
            return 0

        lax.fori_loop(1, SEG // 3 - 1, _mid, 0)
        _slot(SEG - 3, 0, sref, dref, True, sref, SEG - 1)
        if not last:
            if g > 0:
                pltpu.make_async_copy(src_hbm.at[c, s, g + 1], srefn,
                                      si).wait()
                pltpu.make_async_copy(dst_hbm.at[s, g + 1], drefn, si).wait()
            _slot(SEG - 2, 1, sref, dref, True, srefn, 0)
            _slot(SEG - 1, 2, sref, dref, True, srefn, 1)
        else:
            _slot(SEG - 2, 1, sref, dref, True)
            _slot(SEG - 1, 2, sref, dref, True)

    lastd = dst_i0 if (nseg - 1) % 2 == 0 else dst_i1
    pltpu.make_async_copy(rows[2], acc_sh.at[lastd.at[SEG - 1]], ss[2]).wait()

    pltpu.sync_copy(deg_v, deg_out.at[c * NS + s])
    plsc.subcore_barrier()

    for i in range(ZPT // CHUNK):
        r0 = s * ZPT + i * CHUNK
        pltpu.sync_copy(acc_sh.at[pl.ds(r0, CHUNK)], rows_a)
        pltpu.sync_copy(rows_a, acc_out.at[c, pl.ds(r0, CHUNK)])


def _sc_aggregate(x_halves, src5, dst4):
    mesh = plsc.VectorSubcoreMesh(core_axis_name="c", subcore_axis_name="s")
    return pl.kernel(
        _sc_body,
        out_type=(
            jax.ShapeDtypeStruct((NC, R_ACC, DH), jnp.float32),
            jax.ShapeDtypeStruct((NW, R_ACC), jnp.float32),
        ),
        mesh=mesh,
        compiler_params=pltpu.CompilerParams(needs_layout_passes=False,
                                             use_tc_tiling_on_sc=False),
        scratch_types=[
            pltpu.VMEM((SEG, CHUNK), jnp.int32),
            pltpu.VMEM((SEG, CHUNK), jnp.int32),
            pltpu.VMEM((SEG, CHUNK), jnp.int32),
            pltpu.VMEM((SEG, CHUNK), jnp.int32),
            pltpu.VMEM((CHUNK, DH), jnp.float32),
            pltpu.VMEM((CHUNK, DH), jnp.float32),
            pltpu.VMEM((CHUNK, DH), jnp.float32),
            pltpu.VMEM((R_ACC,), jnp.float32),
            pltpu.VMEM_SHARED((R_ACC, DH), jnp.float32),
            pltpu.SemaphoreType.DMA,
            pltpu.SemaphoreType.DMA,
            pltpu.SemaphoreType.DMA,
            pltpu.SemaphoreType.DMA,
            pltpu.SemaphoreType.DMA,
            pltpu.SemaphoreType.DMA,
            pltpu.SemaphoreType.DMA,
        ],
    )(x_halves, src5, dst4)


def _ent_body(x_ref, acc_ref, deg_ref, wskip_ref, wee_ref, wtrans_ref,
              bskip_ref, bee_ref, btrans_ref, out_ref):
    d = jnp.sum(deg_ref[...], axis=0) * 0.5
    d = jnp.maximum(d, 1.0)
    r = 1.0 / d[:, None]
    x = x_ref[...]
    h = jnp.dot(x, wskip_ref[...], preferred_element_type=jnp.float32)
    h = h + jnp.dot(acc_ref[0] * r, wee_ref[:DH, :],
                    preferred_element_type=jnp.float32)
    h = h + jnp.dot(acc_ref[1] * r, wee_ref[DH:, :],
                    preferred_element_type=jnp.float32)
    h = h + bskip_ref[...] + bee_ref[...]
    h = jnp.where(h >= 0, h, 0.01 * h)
    out_ref[...] = jnp.dot(h, wtrans_ref[...],
                           preferred_element_type=jnp.float32) + btrans_ref[...]


def _ent_path(x_entity, acc, deg4, w_skip, w_ee, w_trans, b_skip, b_ee, b_trans):
    blk = ENT_BLK
    grid = R_ACC // blk
    wspec = pl.BlockSpec((D, D), lambda i: (0, 0))
    bspec = pl.BlockSpec((1, D), lambda i: (0, 0))
    return pl.pallas_call(
        _ent_body,
        grid=(grid,),
        in_specs=[
            pl.BlockSpec((blk, D), lambda i: (i, 0)),
            pl.BlockSpec((NC, blk, DH), lambda i: (0, i, 0)),
            pl.BlockSpec((NW, blk), lambda i: (0, i)),
            wspec, wspec, wspec,
            bspec, bspec, bspec,
        ],
        out_specs=pl.BlockSpec((blk, D), lambda i: (i, 0)),
        out_shape=jax.ShapeDtypeStruct((R_ACC, D), jnp.float32),
    )(x_entity, acc, deg4, w_skip, w_ee, w_trans,
      b_skip.reshape(1, D), b_ee.reshape(1, D), b_trans.reshape(1, D))


def _snap_body(xs_ref, acc_ref, deg_ref, ss_src_ref, ss_dst_ref,
               wskip_ref, wes_ref, wss_ref, wtrans_ref,
               bskip_ref, bes_ref, bss_ref, btrans_ref, out_ref):
    m = 16
    rowid = lax.broadcasted_iota(jnp.int32, (m, D), 0)
    rowidh = lax.broadcasted_iota(jnp.int32, (m, DH), 0)
    ds_ = jnp.sum(deg_ref[...], axis=0) * 0.5
    ds_ = jnp.maximum(ds_, 1.0)
    r = 1.0 / ds_[:, None]
    aggl = jnp.where(rowidh < N_SNAP, acc_ref[0], 0.0) * r
    aggr = jnp.where(rowidh < N_SNAP, acc_ref[1], 0.0) * r
    conv_es = (jnp.dot(aggl, wes_ref[:DH, :],
                       preferred_element_type=jnp.float32)
               + jnp.dot(aggr, wes_ref[DH:, :],
                         preferred_element_type=jnp.float32)) + bes_ref[...]
    xs = xs_ref[...]
    h0 = jnp.dot(xs, wskip_ref[...],
                 preferred_element_type=jnp.float32) + bskip_ref[...]
    h0 = h0 + conv_es[:N_SNAP]

    colid = lax.broadcasted_iota(jnp.int32, (m, D), 1)
    a = jnp.zeros((m, D), jnp.float32)
    for e in range(ss_src_ref.shape[0]):
        se = ss_src_ref[e]
        de = ss_dst_ref[e]
        a = a + jnp.where((rowid == de) & (colid == se), 1.0, 0.0)
    h0p = jnp.concatenate([h0, jnp.zeros((D - N_SNAP, D), jnp.float32)], axis=0)
    aggss = jnp.dot(a, h0p, preferred_element_type=jnp.float32)
    degss = jnp.maximum(jnp.sum(a, axis=1), 1.0)
    hs = jnp.dot(aggss / degss[:, None], wss_ref[...],
                 preferred_element_type=jnp.float32) + bss_ref[...]
    hs = jnp.where(hs >= 0, hs, 0.01 * hs)
    res = jnp.dot(hs, wtrans_ref[...],
                  preferred_element_type=jnp.float32) + btrans_ref[...]
    out_ref[...] = res[:N_SNAP]


def _snap_path(x_snapshot, acc_es, deg_es, ss_src, ss_dst,
               w_skip, w_es, w_ss, w_trans, b_skip, b_es, b_ss, b_trans):
    wspec = pl.BlockSpec((D, D), lambda: (0, 0))
    bspec = pl.BlockSpec((1, D), lambda: (0, 0))
    sspec = pl.BlockSpec(memory_space=pltpu.SMEM)
    return pl.pallas_call(
        _snap_body,
        in_specs=[
            pl.BlockSpec((N_SNAP, D), lambda: (0, 0)),
            pl.BlockSpec((NC, 16, DH), lambda: (0, 0, 0)),
            pl.BlockSpec((NW, 16), lambda: (0, 0)),
            sspec, sspec,
            wspec, wspec, wspec, wspec,
            bspec, bspec, bspec, bspec,
        ],
        out_specs=pl.BlockSpec((N_SNAP, D), lambda: (0, 0)),
        out_shape=jax.ShapeDtypeStruct((N_SNAP, D), jnp.float32),
    )(x_snapshot, acc_es, deg_es, ss_src, ss_dst,
      w_skip, w_es, w_ss, w_trans,
      b_skip.reshape(1, D), b_es.reshape(1, D), b_ss.reshape(1, D),
      b_trans.reshape(1, D))


def kernel(x_entity, x_snapshot, ee_src, ee_dst, es_src, es_dst, ss_src, ss_dst,
           W_ee, b_ee, W_es, b_es, W_ss, b_ss,
           W_skip_ent, b_skip_ent, W_skip_snap, b_skip_snap,
           W_trans_ent, b_trans_ent, W_trans_snap, b_trans_snap):
    e_total = ee_src.shape[0] + es_src.shape[0]
    e_pad = -(-e_total // (NS * CHUNK * SEG)) * (NS * CHUNK * SEG)
    npad = e_pad - e_total
    xh = x_entity.reshape(2 * N_ENT, DH)
    srcg = jnp.concatenate([ee_src, es_src, jnp.zeros((npad,), jnp.int32)])
    src5 = jnp.stack([srcg * 2, srcg * 2 + 1]).reshape(NC, NS, -1, SEG, CHUNK)
    padrows = DUMMY + (jnp.arange(npad, dtype=jnp.int32) & 127)
    dstg = jnp.concatenate([ee_dst, es_dst + N_ENT, padrows])
    dst4 = dstg.reshape(NS, -1, SEG, CHUNK)

    acc, deg = _sc_aggregate(xh, src5, dst4)

    xp = jnp.concatenate(
        [x_entity, jnp.zeros((R_ACC - N_ENT, D), jnp.float32)])
    h_ent = _ent_path(xp, acc, deg, W_skip_ent, W_ee, W_trans_ent,
                      b_skip_ent, b_ee, b_trans_ent)[:N_ENT]

    acc_es = acc[:, N_ENT:N_ENT + 16, :]
    deg_es = deg[:, N_ENT:N_ENT + 16]
    h_snap = _snap_path(x_snapshot, acc_es, deg_es, ss_src, ss_dst,
                        W_skip_snap, W_es, W_ss, W_trans_snap,
                        b_skip_snap, b_es, b_ss, b_trans_snap)
    return (h_ent, h_snap)


# speedup vs baseline: 8.6282x; 1.0659x over previous
import jax
import jax.numpy as jnp
from jax import lax
from jax.experimental import pallas as pl
from jax.experimental.pallas import tpu as pltpu
from jax.experimental.pallas import tpu_sc as plsc

N_ENT = 10000
N_SNAP = 10
D = 128
DH = D // 2

NC = 2
NS = 16
NW = NC * NS
LANES = 16
CHUNK = 128
SEG = 27

R_ACC = 10240
DUMMY = N_ENT + N_SNAP
ZPT = R_ACC // NS
ENT_BLK = 2048


def _sc_body(x_hbm, src_hbm, dst_hbm,
             acc_out, deg_out,
             src_i0, src_i1, dst_i0, dst_i1,
             rows_a, rows_b, rows_c, deg_v,
             acc_sh, sg0, sg1, sg2, ss0, ss1, ss2, si):
    c = lax.axis_index("c")
    s = lax.axis_index("s")
    nseg = dst_hbm.shape[1]
    rows = (rows_a, rows_b, rows_c)
    sg = (sg0, sg1, sg2)
    ss = (ss0, ss1, ss2)

    zeros16 = jnp.zeros((LANES,), jnp.float32)
    ones16 = jnp.ones((LANES,), jnp.float32)

    def _zrow(i, _):
        for k in range(DH // LANES):
            rows_a[i, pl.ds(k * LANES, LANES)] = zeros16
        return 0
    lax.fori_loop(0, CHUNK, _zrow, 0)

    def _zdeg(i, _):
        deg_v[pl.ds(i * LANES, LANES)] = zeros16
        return 0
    lax.fori_loop(0, R_ACC // LANES, _zdeg, 0)

    for i in range(ZPT // CHUNK):
        pltpu.sync_copy(rows_a, acc_sh.at[pl.ds(s * ZPT + i * CHUNK, CHUNK)])

    def _fix_src(ref):
        def _b(r, _):
            for k in range(CHUNK // LANES):
                sl = ref[r, pl.ds(k * LANES, LANES)]
                ref[r, pl.ds(k * LANES, LANES)] = sl * 2 + c
            return 0
        lax.fori_loop(0, SEG, _b, 0)

    pltpu.sync_copy(src_hbm.at[s, 0], src_i0)
    pltpu.sync_copy(dst_hbm.at[s, 0], dst_i0)
    pltpu.sync_copy(src_hbm.at[s, 1], src_i1)
    pltpu.sync_copy(dst_hbm.at[s, 1], dst_i1)
    _fix_src(src_i0)
    _fix_src(src_i1)
    plsc.subcore_barrier()

    def _deg_update(dref, lj):
        for k in range(CHUNK // LANES):
            idx = dref[lj, pl.ds(k * LANES, LANES)]
            plsc.addupdate_scatter(deg_v, [idx], ones16)

    def _slot(lj, k, sref, dref, wait_s, gref=None, glj=None):
        k2 = (k + 2) % 3
        pltpu.make_async_copy(x_hbm.at[sref.at[lj]], rows[k], sg[k]).wait()
        _deg_update(dref, lj)
        pltpu.async_copy(rows[k], acc_sh.at[dref.at[lj]], ss[k], add=True)
        if wait_s:
            pltpu.make_async_copy(rows[k2], acc_sh.at[dref.at[lj]],
                                  ss[k2]).wait()
        if gref is not None:
            pltpu.async_copy(x_hbm.at[gref.at[glj]], rows[k2], sg[k2])

    pltpu.async_copy(x_hbm.at[src_i0.at[0]], rows_a, sg0)
    pltpu.async_copy(x_hbm.at[src_i0.at[1]], rows_b, sg1)

    for g in range(nseg):
        if g % 2 == 0:
            sref, dref, srefn, drefn = src_i0, dst_i0, src_i1, dst_i1
        else:
            sref, dref, srefn, drefn = src_i1, dst_i1, src_i0, dst_i0
        last = g == nseg - 1
        _slot(0, 0, sref, dref, wait_s=(g > 0), gref=sref, glj=2)
        if 0 < g < nseg - 1:
            pltpu.async_copy(src_hbm.at[s, g + 1], srefn, si)
            pltpu.async_copy(dst_hbm.at[s, g + 1], drefn, si)
        _slot(1, 1, sref, dref, True, sref, 3)
        _slot(2, 2, sref, dref, True, sref, 4)

        def _mid(t, _):
            l0 = 3 * t
            _slot(l0, 0, sref, dref, True, sref, l0 + 2)
            _slot(l0 + 1, 1, sref, dref, True, sref, l0 + 3)
            _slot(l0 + 2, 2, sref, dref, True, sref, l0 + 4)
            return 0

        lax.fori_loop(1, SEG // 3 - 1, _mid, 0)
        _slot(SEG - 3, 0, sref, dref, True, sref, SEG - 1)
        if not last:
            if g > 0:
                pltpu.make_async_copy(src_hbm.at[s, g + 1], srefn, si).wait()
                pltpu.make_async_copy(dst_hbm.at[s, g + 1], drefn, si).wait()
                _fix_src(srefn)
            _slot(SEG - 2, 1, sref, dref, True, srefn, 0)
            _slot(SEG - 1, 2, sref, dref, True, srefn, 1)
        else:
            _slot(SEG - 2, 1, sref, dref, True)
            _slot(SEG - 1, 2, sref, dref, True)

    lastd = dst_i0 if (nseg - 1) % 2 == 0 else dst_i1
    pltpu.make_async_copy(rows[2], acc_sh.at[lastd.at[SEG - 1]], ss[2]).wait()

    pltpu.sync_copy(deg_v, deg_out.at[c * NS + s])
    plsc.subcore_barrier()

    for i in range(ZPT // CHUNK):
        r0 = s * ZPT + i * CHUNK
        pltpu.sync_copy(acc_sh.at[pl.ds(r0, CHUNK)], rows_a)
        pltpu.sync_copy(rows_a, acc_out.at[c, pl.ds(r0, CHUNK)])


def _sc_aggregate(x_halves, src5, dst4):
    mesh = plsc.VectorSubcoreMesh(core_axis_name="c", subcore_axis_name="s")
    return pl.kernel(
        _sc_body,
        out_type=(
            jax.ShapeDtypeStruct((NC, R_ACC, DH), jnp.float32),
            jax.ShapeDtypeStruct((NW, R_ACC), jnp.float32),
        ),
        mesh=mesh,
        compiler_params=pltpu.CompilerParams(needs_layout_passes=False,
                                             use_tc_tiling_on_sc=False),
        scratch_types=[
            pltpu.VMEM((SEG, CHUNK), jnp.int32),
            pltpu.VMEM((SEG, CHUNK), jnp.int32),
            pltpu.VMEM((SEG, CHUNK), jnp.int32),
            pltpu.VMEM((SEG, CHUNK), jnp.int32),
            pltpu.VMEM((CHUNK, DH), jnp.float32),
            pltpu.VMEM((CHUNK, DH), jnp.float32),
            pltpu.VMEM((CHUNK, DH), jnp.float32),
            pltpu.VMEM((R_ACC,), jnp.float32),
            pltpu.VMEM_SHARED((R_ACC, DH), jnp.float32),
            pltpu.SemaphoreType.DMA,
            pltpu.SemaphoreType.DMA,
            pltpu.SemaphoreType.DMA,
            pltpu.SemaphoreType.DMA,
            pltpu.SemaphoreType.DMA,
            pltpu.SemaphoreType.DMA,
            pltpu.SemaphoreType.DMA,
        ],
    )(x_halves, src5, dst4)


def _ent_body(x_ref, acc_ref, deg_ref, wskip_ref, wee_ref, wtrans_ref,
              bskip_ref, bee_ref, btrans_ref, out_ref):
    d = jnp.sum(deg_ref[...], axis=0) * 0.5
    d = jnp.maximum(d, 1.0)
    r = 1.0 / d[:, None]
    x = x_ref[...]
    h = jnp.dot(x, wskip_ref[...], preferred_element_type=jnp.float32)
    h = h + jnp.dot(acc_ref[0] * r, wee_ref[:DH, :],
                    preferred_element_type=jnp.float32)
    h = h + jnp.dot(acc_ref[1] * r, wee_ref[DH:, :],
                    preferred_element_type=jnp.float32)
    h = h + bskip_ref[...] + bee_ref[...]
    h = jnp.where(h >= 0, h, 0.01 * h)
    out_ref[...] = jnp.dot(h, wtrans_ref[...],
                           preferred_element_type=jnp.float32) + btrans_ref[...]


def _ent_path(x_entity, acc, deg4, w_skip, w_ee, w_trans, b_skip, b_ee, b_trans):
    blk = ENT_BLK
    grid = -(-N_ENT // blk)
    wspec = pl.BlockSpec((D, D), lambda i: (0, 0))
    bspec = pl.BlockSpec((1, D), lambda i: (0, 0))
    return pl.pallas_call(
        _ent_body,
        grid=(grid,),
        in_specs=[
            pl.BlockSpec((blk, D), lambda i: (i, 0)),
            pl.BlockSpec((NC, blk, DH), lambda i: (0, i, 0)),
            pl.BlockSpec((NW, blk), lambda i: (0, i)),
            wspec, wspec, wspec,
            bspec, bspec, bspec,
        ],
        out_specs=pl.BlockSpec((blk, D), lambda i: (i, 0)),
        out_shape=jax.ShapeDtypeStruct((N_ENT, D), jnp.float32),
    )(x_entity, acc, deg4, w_skip, w_ee, w_trans,
      b_skip.reshape(1, D), b_ee.reshape(1, D), b_trans.reshape(1, D))


def _snap_body(xs_ref, acc_ref, deg_ref, ss_src_ref, ss_dst_ref,
               wskip_ref, wes_ref, wss_ref, wtrans_ref,
               bskip_ref, bes_ref, bss_ref, btrans_ref, out_ref):
    m = 16
    rowid = lax.broadcasted_iota(jnp.int32, (m, D), 0)
    rowidh = lax.broadcasted_iota(jnp.int32, (m, DH), 0)
    ds_ = jnp.sum(deg_ref[...], axis=0) * 0.5
    ds_ = jnp.maximum(ds_, 1.0)
    r = 1.0 / ds_[:, None]
    aggl = jnp.where(rowidh < N_SNAP, acc_ref[0], 0.0) * r
    aggr = jnp.where(rowidh < N_SNAP, acc_ref[1], 0.0) * r
    conv_es = (jnp.dot(aggl, wes_ref[:DH, :],
                       preferred_element_type=jnp.float32)
               + jnp.dot(aggr, wes_ref[DH:, :],
                         preferred_element_type=jnp.float32)) + bes_ref[...]
    xs = xs_ref[...]
    h0 = jnp.dot(xs, wskip_ref[...],
                 preferred_element_type=jnp.float32) + bskip_ref[...]
    h0 = h0 + conv_es[:N_SNAP]

    colid = lax.broadcasted_iota(jnp.int32, (m, D), 1)
    a = jnp.zeros((m, D), jnp.float32)
    for e in range(ss_src_ref.shape[0]):
        se = ss_src_ref[e]
        de = ss_dst_ref[e]
        a = a + jnp.where((rowid == de) & (colid == se), 1.0, 0.0)
    h0p = jnp.concatenate([h0, jnp.zeros((D - N_SNAP, D), jnp.float32)], axis=0)
    aggss = jnp.dot(a, h0p, preferred_element_type=jnp.float32)
    degss = jnp.maximum(jnp.sum(a, axis=1), 1.0)
    hs = jnp.dot(aggss / degss[:, None], wss_ref[...],
                 preferred_element_type=jnp.float32) + bss_ref[...]
    hs = jnp.where(hs >= 0, hs, 0.01 * hs)
    res = jnp.dot(hs, wtrans_ref[...],
                  preferred_element_type=jnp.float32) + btrans_ref[...]
    out_ref[...] = res[:N_SNAP]


def _snap_path(x_snapshot, acc_es, deg_es, ss_src, ss_dst,
               w_skip, w_es, w_ss, w_trans, b_skip, b_es, b_ss, b_trans):
    wspec = pl.BlockSpec((D, D), lambda: (0, 0))
    bspec = pl.BlockSpec((1, D), lambda: (0, 0))
    sspec = pl.BlockSpec(memory_space=pltpu.SMEM)
    return pl.pallas_call(
        _snap_body,
        in_specs=[
            pl.BlockSpec((N_SNAP, D), lambda: (0, 0)),
            pl.BlockSpec((NC, 16, DH), lambda: (0, 0, 0)),
            pl.BlockSpec((NW, 16), lambda: (0, 0)),
            sspec, sspec,
            wspec, wspec, wspec, wspec,
            bspec, bspec, bspec, bspec,
        ],
        out_specs=pl.BlockSpec((N_SNAP, D), lambda: (0, 0)),
        out_shape=jax.ShapeDtypeStruct((N_SNAP, D), jnp.float32),
    )(x_snapshot, acc_es, deg_es, ss_src, ss_dst,
      w_skip, w_es, w_ss, w_trans,
      b_skip.reshape(1, D), b_es.reshape(1, D), b_ss.reshape(1, D),
      b_trans.reshape(1, D))


def kernel(x_entity, x_snapshot, ee_src, ee_dst, es_src, es_dst, ss_src, ss_dst,
           W_ee, b_ee, W_es, b_es, W_ss, b_ss,
           W_skip_ent, b_skip_ent, W_skip_snap, b_skip_snap,
           W_trans_ent, b_trans_ent, W_trans_snap, b_trans_snap):
    e_total = ee_src.shape[0] + es_src.shape[0]
    e_pad = -(-e_total // (NS * CHUNK * SEG)) * (NS * CHUNK * SEG)
    npad = e_pad - e_total
    xh = x_entity.reshape(2 * N_ENT, DH)
    srcg = jnp.concatenate([ee_src, es_src, jnp.zeros((npad,), jnp.int32)])
    src5 = srcg.reshape(NS, -1, SEG, CHUNK)
    padrows = DUMMY + (jnp.arange(npad, dtype=jnp.int32) & 127)
    dstg = jnp.concatenate([ee_dst, es_dst + N_ENT, padrows])
    dst4 = dstg.reshape(NS, -1, SEG, CHUNK)

    acc, deg = _sc_aggregate(xh, src5, dst4)

    h_ent = _ent_path(x_entity, acc, deg, W_skip_ent, W_ee, W_trans_ent,
                      b_skip_ent, b_ee, b_trans_ent)

    acc_es = acc[:, N_ENT:N_ENT + 16, :]
    deg_es = deg[:, N_ENT:N_ENT + 16]
    h_snap = _snap_path(x_snapshot, acc_es, deg_es, ss_src, ss_dst,
                        W_skip_snap, W_es, W_ss, W_trans_snap,
                        b_skip_snap, b_es, b_ss, b_trans_snap)
    return (h_ent, h_snap)


# speedup vs baseline: 8.8323x; 1.0236x over previous
import jax
import jax.numpy as jnp
from jax import lax
from jax.experimental import pallas as pl
from jax.experimental.pallas import tpu as pltpu
from jax.experimental.pallas import tpu_sc as plsc

N_ENT = 10000
N_SNAP = 10
D = 128
DH = D // 2

NC = 2
NS = 16
NW = NC * NS
LANES = 16
CHUNK = 128
SEG = 27

R_ACC = 10240
DUMMY = N_ENT + N_SNAP
ZPT = R_ACC // NS
ENT_BLK = 2048


def _sc_body(x_hbm, src_hbm, dst_hbm,
             acc_out, deg_out,
             src_i0, src_i1, dst_i0, dst_i1,
             rows_a, rows_b, rows_c, deg_v,
             acc_sh, sg0, sg1, sg2, ss0, ss1, ss2, si):
    c = lax.axis_index("c")
    s = lax.axis_index("s")
    nseg = dst_hbm.shape[1]
    rows = (rows_a, rows_b, rows_c)
    sg = (sg0, sg1, sg2)
    ss = (ss0, ss1, ss2)

    zeros16 = jnp.zeros((LANES,), jnp.float32)
    ones16 = jnp.ones((LANES,), jnp.float32)

    def _zrow(i, _):
        for k in range(DH // LANES):
            rows_a[i, pl.ds(k * LANES, LANES)] = zeros16
        return 0
    lax.fori_loop(0, CHUNK, _zrow, 0)

    def _zdeg(i, _):
        deg_v[pl.ds(i * LANES, LANES)] = zeros16
        return 0
    lax.fori_loop(0, R_ACC // LANES, _zdeg, 0)

    for i in range(ZPT // CHUNK):
        pltpu.sync_copy(rows_a, acc_sh.at[pl.ds(s * ZPT + i * CHUNK, CHUNK)])

    def _fix_src(ref):
        def _b(r, _):
            for k in range(CHUNK // LANES):
                sl = ref[r, pl.ds(k * LANES, LANES)]
                ref[r, pl.ds(k * LANES, LANES)] = sl * 2 + c
            return 0
        lax.fori_loop(0, SEG, _b, 0)

    pltpu.sync_copy(src_hbm.at[s, 0], src_i0)
    pltpu.sync_copy(dst_hbm.at[s, 0], dst_i0)
    pltpu.sync_copy(src_hbm.at[s, 1], src_i1)
    pltpu.sync_copy(dst_hbm.at[s, 1], dst_i1)
    _fix_src(src_i0)
    _fix_src(src_i1)
    plsc.subcore_barrier()

    def _deg_update(dref, lj):
        for k in range(CHUNK // LANES):
            idx = dref[lj, pl.ds(k * LANES, LANES)]
            plsc.addupdate_scatter(deg_v, [idx], ones16)

    def _slot(lj, k, sref, dref, wait_s, gref=None, glj=None):
        k2 = (k + 2) % 3
        pltpu.make_async_copy(x_hbm.at[sref.at[lj]], rows[k], sg[k]).wait()
        _deg_update(dref, lj)
        pltpu.async_copy(rows[k], acc_sh.at[dref.at[lj]], ss[k], add=True)
        if wait_s:
            pltpu.make_async_copy(rows[k2], acc_sh.at[dref.at[lj]],
                                  ss[k2]).wait()
        if gref is not None:
            pltpu.async_copy(x_hbm.at[gref.at[glj]], rows[k2], sg[k2])

    pltpu.async_copy(x_hbm.at[src_i0.at[0]], rows_a, sg0)
    pltpu.async_copy(x_hbm.at[src_i0.at[1]], rows_b, sg1)

    for g in range(nseg):
        if g % 2 == 0:
            sref, dref, srefn, drefn = src_i0, dst_i0, src_i1, dst_i1
        else:
            sref, dref, srefn, drefn = src_i1, dst_i1, src_i0, dst_i0
        last = g == nseg - 1
        _slot(0, 0, sref, dref, wait_s=(g > 0), gref=sref, glj=2)
        if 0 < g < nseg - 1:
            pltpu.async_copy(src_hbm.at[s, g + 1], srefn, si)
            pltpu.async_copy(dst_hbm.at[s, g + 1], drefn, si)
        _slot(1, 1, sref, dref, True, sref, 3)
        _slot(2, 2, sref, dref, True, sref, 4)

        def _mid(t, _):
            l0 = 3 * t
            _slot(l0, 0, sref, dref, True, sref, l0 + 2)
            _slot(l0 + 1, 1, sref, dref, True, sref, l0 + 3)
            _slot(l0 + 2, 2, sref, dref, True, sref, l0 + 4)
            return 0

        lax.fori_loop(1, SEG // 3 - 1, _mid, 0)
        _slot(SEG - 3, 0, sref, dref, True, sref, SEG - 1)
        if not last:
            if g > 0:
                pltpu.make_async_copy(src_hbm.at[s, g + 1], srefn, si).wait()
                pltpu.make_async_copy(dst_hbm.at[s, g + 1], drefn, si).wait()
                _fix_src(srefn)
            _slot(SEG - 2, 1, sref, dref, True, srefn, 0)
            _slot(SEG - 1, 2, sref, dref, True, srefn, 1)
        else:
            _slot(SEG - 2, 1, sref, dref, True)
            _slot(SEG - 1, 2, sref, dref, True)

    lastd = dst_i0 if (nseg - 1) % 2 == 0 else dst_i1
    pltpu.make_async_copy(rows[2], acc_sh.at[lastd.at[SEG - 1]], ss[2]).wait()

    pltpu.sync_copy(deg_v, deg_out.at[c * NS + s])
    plsc.subcore_barrier()

    pltpu.sync_copy(acc_sh.at[pl.ds(s * ZPT, ZPT)],
                    acc_out.at[c, pl.ds(s * ZPT, ZPT)])


def _sc_aggregate(x_halves, src5, dst4):
    mesh = plsc.VectorSubcoreMesh(core_axis_name="c", subcore_axis_name="s")
    return pl.kernel(
        _sc_body,
        out_type=(
            jax.ShapeDtypeStruct((NC, R_ACC, DH), jnp.float32),
            jax.ShapeDtypeStruct((NW, R_ACC), jnp.float32),
        ),
        mesh=mesh,
        compiler_params=pltpu.CompilerParams(needs_layout_passes=False,
                                             use_tc_tiling_on_sc=False),
        scratch_types=[
            pltpu.VMEM((SEG, CHUNK), jnp.int32),
            pltpu.VMEM((SEG, CHUNK), jnp.int32),
            pltpu.VMEM((SEG, CHUNK), jnp.int32),
            pltpu.VMEM((SEG, CHUNK), jnp.int32),
            pltpu.VMEM((CHUNK, DH), jnp.float32),
            pltpu.VMEM((CHUNK, DH), jnp.float32),
            pltpu.VMEM((CHUNK, DH), jnp.float32),
            pltpu.VMEM((R_ACC,), jnp.float32),
            pltpu.VMEM_SHARED((R_ACC, DH), jnp.float32),
            pltpu.SemaphoreType.DMA,
            pltpu.SemaphoreType.DMA,
            pltpu.SemaphoreType.DMA,
            pltpu.SemaphoreType.DMA,
            pltpu.SemaphoreType.DMA,
            pltpu.SemaphoreType.DMA,
            pltpu.SemaphoreType.DMA,
        ],
    )(x_halves, src5, dst4)


def _ent_body(x_ref, acc_ref, deg_ref, wskip_ref, wee_ref, wtrans_ref,
              bskip_ref, bee_ref, btrans_ref, out_ref):
    d = jnp.sum(deg_ref[...], axis=0) * 0.5
    d = jnp.maximum(d, 1.0)
    r = 1.0 / d[:, None]
    x = x_ref[...]
    h = jnp.dot(x, wskip_ref[...], preferred_element_type=jnp.float32)
    h = h + jnp.dot(acc_ref[0] * r, wee_ref[:DH, :],
                    preferred_element_type=jnp.float32)
    h = h + jnp.dot(acc_ref[1] * r, wee_ref[DH:, :],
                    preferred_element_type=jnp.float32)
    h = h + bskip_ref[...] + bee_ref[...]
    h = jnp.where(h >= 0, h, 0.01 * h)
    out_ref[...] = jnp.dot(h, wtrans_ref[...],
                           preferred_element_type=jnp.float32) + btrans_ref[...]


def _ent_path(x_entity, acc, deg4, w_skip, w_ee, w_trans, b_skip, b_ee, b_trans):
    blk = ENT_BLK
    grid = -(-N_ENT // blk)
    wspec = pl.BlockSpec((D, D), lambda i: (0, 0))
    bspec = pl.BlockSpec((1, D), lambda i: (0, 0))
    return pl.pallas_call(
        _ent_body,
        grid=(grid,),
        in_specs=[
            pl.BlockSpec((blk, D), lambda i: (i, 0)),
            pl.BlockSpec((NC, blk, DH), lambda i: (0, i, 0)),
            pl.BlockSpec((NW, blk), lambda i: (0, i)),
            wspec, wspec, wspec,
            bspec, bspec, bspec,
        ],
        out_specs=pl.BlockSpec((blk, D), lambda i: (i, 0)),
        out_shape=jax.ShapeDtypeStruct((N_ENT, D), jnp.float32),
    )(x_entity, acc, deg4, w_skip, w_ee, w_trans,
      b_skip.reshape(1, D), b_ee.reshape(1, D), b_trans.reshape(1, D))


def _snap_body(xs_ref, acc_ref, deg_ref, ss_src_ref, ss_dst_ref,
               wskip_ref, wes_ref, wss_ref, wtrans_ref,
               bskip_ref, bes_ref, bss_ref, btrans_ref, out_ref):
    m = 16
    rowid = lax.broadcasted_iota(jnp.int32, (m, D), 0)
    rowidh = lax.broadcasted_iota(jnp.int32, (m, DH), 0)
    ds_ = jnp.sum(deg_ref[...], axis=0) * 0.5
    ds_ = jnp.maximum(ds_, 1.0)
    r = 1.0 / ds_[:, None]
    aggl = jnp.where(rowidh < N_SNAP, acc_ref[0], 0.0) * r
    aggr = jnp.where(rowidh < N_SNAP, acc_ref[1], 0.0) * r
    conv_es = (jnp.dot(aggl, wes_ref[:DH, :],
                       preferred_element_type=jnp.float32)
               + jnp.dot(aggr, wes_ref[DH:, :],
                         preferred_element_type=jnp.float32)) + bes_ref[...]
    xs = xs_ref[...]
    h0 = jnp.dot(xs, wskip_ref[...],
                 preferred_element_type=jnp.float32) + bskip_ref[...]
    h0 = h0 + conv_es[:N_SNAP]

    colid = lax.broadcasted_iota(jnp.int32, (m, D), 1)
    a = jnp.zeros((m, D), jnp.float32)
    for e in range(ss_src_ref.shape[0]):
        se = ss_src_ref[e]
        de = ss_dst_ref[e]
        a = a + jnp.where((rowid == de) & (colid == se), 1.0, 0.0)
    h0p = jnp.concatenate([h0, jnp.zeros((D - N_SNAP, D), jnp.float32)], axis=0)
    aggss = jnp.dot(a, h0p, preferred_element_type=jnp.float32)
    degss = jnp.maximum(jnp.sum(a, axis=1), 1.0)
    hs = jnp.dot(aggss / degss[:, None], wss_ref[...],
                 preferred_element_type=jnp.float32) + bss_ref[...]
    hs = jnp.where(hs >= 0, hs, 0.01 * hs)
    res = jnp.dot(hs, wtrans_ref[...],
                  preferred_element_type=jnp.float32) + btrans_ref[...]
    out_ref[...] = res[:N_SNAP]


def _snap_path(x_snapshot, acc_es, deg_es, ss_src, ss_dst,
               w_skip, w_es, w_ss, w_trans, b_skip, b_es, b_ss, b_trans):
    wspec = pl.BlockSpec((D, D), lambda: (0, 0))
    bspec = pl.BlockSpec((1, D), lambda: (0, 0))
    sspec = pl.BlockSpec(memory_space=pltpu.SMEM)
    return pl.pallas_call(
        _snap_body,
        in_specs=[
            pl.BlockSpec((N_SNAP, D), lambda: (0, 0)),
            pl.BlockSpec((NC, 16, DH), lambda: (0, 0, 0)),
            pl.BlockSpec((NW, 16), lambda: (0, 0)),
            sspec, sspec,
            wspec, wspec, wspec, wspec,
            bspec, bspec, bspec, bspec,
        ],
        out_specs=pl.BlockSpec((N_SNAP, D), lambda: (0, 0)),
        out_shape=jax.ShapeDtypeStruct((N_SNAP, D), jnp.float32),
    )(x_snapshot, acc_es, deg_es, ss_src, ss_dst,
      w_skip, w_es, w_ss, w_trans,
      b_skip.reshape(1, D), b_es.reshape(1, D), b_ss.reshape(1, D),
      b_trans.reshape(1, D))


def kernel(x_entity, x_snapshot, ee_src, ee_dst, es_src, es_dst, ss_src, ss_dst,
           W_ee, b_ee, W_es, b_es, W_ss, b_ss,
           W_skip_ent, b_skip_ent, W_skip_snap, b_skip_snap,
           W_trans_ent, b_trans_ent, W_trans_snap, b_trans_snap):
    n_ee = ee_src.shape[0]
    n_es = es_src.shape[0]
    es_off = -(-n_ee // 1024) * 1024
    e_total = es_off + n_es
    e_pad = -(-e_total // (NS * CHUNK * SEG)) * (NS * CHUNK * SEG)
    xh = x_entity.reshape(2 * N_ENT, DH)
    srcg = jnp.zeros((e_pad,), jnp.int32)
    srcg = jax.lax.dynamic_update_slice(srcg, ee_src, (0,))
    srcg = jax.lax.dynamic_update_slice(srcg, es_src, (es_off,))
    src5 = srcg.reshape(NS, -1, SEG, CHUNK)
    dstg = DUMMY + (jnp.arange(e_pad, dtype=jnp.int32) & 127)
    dstg = jax.lax.dynamic_update_slice(dstg, ee_dst, (0,))
    dstg = jax.lax.dynamic_update_slice(dstg, es_dst + N_ENT, (es_off,))
    dst4 = dstg.reshape(NS, -1, SEG, CHUNK)

    acc, deg = _sc_aggregate(xh, src5, dst4)

    h_ent = _ent_path(x_entity, acc, deg, W_skip_ent, W_ee, W_trans_ent,
                      b_skip_ent, b_ee, b_trans_ent)

    acc_es = acc[:, N_ENT:N_ENT + 16, :]
    deg_es = deg[:, N_ENT:N_ENT + 16]
    h_snap = _snap_path(x_snapshot, acc_es, deg_es, ss_src, ss_dst,
                        W_skip_snap, W_es, W_ss, W_trans_snap,
                        b_skip_snap, b_es, b_ss, b_trans_snap)
    return (h_ent, h_snap)
